# baseline (speedup 1.0000x reference)
import jax
import jax.numpy as jnp
from jax.experimental import pallas as pl
from jax.experimental.pallas import tpu as pltpu

_B = 4
_N = 514
_P = 16
_HID = 64
_NH = 4
_HD = 16
_NC = 3
_ALPHA = 0.5
_NEG = -1e9


def _enc_kernel(nf_ref, edges_ref, wemb_ref,
                wkt1_ref, asrc1_ref, adstT1_ref, wo1_ref,
                wkt2_ref, asrc2_ref, adstT2_ref, wo2_ref,
                h_ref, hm_ref, ec_ref):
    x = nf_ref[0]
    h = jnp.dot(x, wemb_ref[...], preferred_element_type=jnp.float32)
    maskf = (edges_ref[0] > 0).astype(jnp.bfloat16)

    ones_col = jnp.ones((_N, 1), jnp.float32)
    lane64 = jax.lax.broadcasted_iota(jnp.int32, (1, _HID), 1)
    headmasks = [((lane64 >= hh * _HD) & (lane64 < (hh + 1) * _HD))
                 .astype(jnp.float32) for hh in range(_NH)]

    def gcrn(h, wkt_ref, asrc_ref, adstT_ref, wo_ref):
        for layer in range(2):
            acc = jnp.zeros((_N, _HID), jnp.float32)
            for c in range(_NC):
                hp = jnp.dot(h, wkt_ref[c],
                             preferred_element_type=jnp.float32)
                hpT = hp.T
                s_src = jnp.dot(hp, asrc_ref[c],
                                preferred_element_type=jnp.float32)
                s_dst = jnp.dot(adstT_ref[c], hpT,
                                preferred_element_type=jnp.float32)
                gc = jnp.exp(0.8 * s_src).astype(jnp.bfloat16)
                e1r = jnp.exp(s_dst).astype(jnp.bfloat16)
                e2r = jnp.exp(0.2 * s_dst).astype(jnp.bfloat16)
                hp_aug = jnp.concatenate([hp, ones_col],
                                         axis=1).astype(jnp.bfloat16)
                for hh in range(_NH):
                    m1 = gc[:, hh:hh + 1] * e1r[hh:hh + 1, :]
                    e = jnp.maximum(m1, e2r[hh:hh + 1, :]) * maskf[c]
                    oo = jnp.dot(e, hp_aug,
                                 preferred_element_type=jnp.float32)
                    den = oo[:, _HID:_HID + 1]
                    recip = jnp.where(den > 0, 1.0 / den, 0.0)
                    acc = acc + oo[:, :_HID] * (recip * headmasks[hh])
            o = jnp.dot(acc * (1.0 / _NC), wo_ref[...],
                        preferred_element_type=jnp.float32)
            h = _ALPHA * h + (1.0 - _ALPHA) * jnp.maximum(o, 0.0)
        return h

    h = gcrn(h, wkt1_ref, asrc1_ref, adstT1_ref, wo1_ref)
    h = gcrn(h, wkt2_ref, asrc2_ref, adstT2_ref, wo2_ref)

    ec_sums = [jnp.sum(jnp.max(maskf[c], axis=1, keepdims=True)
                       .astype(jnp.float32)) * float(_NH)
               for c in range(_NC)]

    h_ref[0] = h
    hm_ref[0] = jnp.mean(h, axis=0, keepdims=True)
    lane = jax.lax.broadcasted_iota(jnp.int32, (1, 128), 1)
    scale = 1.0 / (_NH * _N * _N)
    row = jnp.zeros((1, 128), jnp.float32)
    for c in range(_NC):
        row = row + jnp.where(lane == c, ec_sums[c] * scale, 0.0)
    ec_ref[0] = row


def _prep(Wk, a):
    eye = jnp.eye(_NH, dtype=jnp.float32)
    wkt = Wk.transpose(0, 2, 1, 3).reshape(_NC, _HID, _NH * _HD)
    asrc = (a[..., :_HD][:, :, :, None] * eye[:, None, :]).reshape(
        _NC, _NH * _HD, _NH)
    adstT = (a[..., _HD:][:, :, None, :] * eye[:, :, None]).reshape(
        _NC, _NH, _NH * _HD)
    return wkt, asrc, adstT


@jax.jit
def kernel(node_features, heterogeneous_edges, W_emb, Wk1, a1, Wo1,
           Wk2, a2, Wo2):
    wkt1, asrc1, adstT1 = _prep(Wk1, a1)
    wkt2, asrc2, adstT2 = _prep(Wk2, a2)

    full3 = lambda b: (0, 0, 0)
    full2 = lambda b: (0, 0)
    h_full, hm, ec_pad = pl.pallas_call(
        _enc_kernel,
        grid=(_B,),
        in_specs=[
            pl.BlockSpec((1, _N, _P), lambda b: (b, 0, 0)),
            pl.BlockSpec((1, _NC, _N, _N), lambda b: (b, 0, 0, 0)),
            pl.BlockSpec((_P, _HID), full2),
            pl.BlockSpec((_NC, _HID, _NH * _HD), full3),
            pl.BlockSpec((_NC, _NH * _HD, _NH), full3),
            pl.BlockSpec((_NC, _NH, _NH * _HD), full3),
            pl.BlockSpec((_HID, _HID), full2),
            pl.BlockSpec((_NC, _HID, _NH * _HD), full3),
            pl.BlockSpec((_NC, _NH * _HD, _NH), full3),
            pl.BlockSpec((_NC, _NH, _NH * _HD), full3),
            pl.BlockSpec((_HID, _HID), full2),
        ],
        out_specs=[
            pl.BlockSpec((1, _N, _HID), lambda b: (b, 0, 0)),
            pl.BlockSpec((1, 1, _HID), lambda b: (b, 0, 0)),
            pl.BlockSpec((1, 1, 128), lambda b: (b, 0, 0)),
        ],
        out_shape=[
            jax.ShapeDtypeStruct((_B, _N, _HID), jnp.float32),
            jax.ShapeDtypeStruct((_B, 1, _HID), jnp.float32),
            jax.ShapeDtypeStruct((_B, 1, 128), jnp.float32),
        ],
        compiler_params=pltpu.CompilerParams(
            dimension_semantics=("parallel",)),
    )(node_features, heterogeneous_edges, W_emb,
      wkt1, asrc1, adstT1, Wo1, wkt2, asrc2, adstT2, Wo2)

    return hm[:, 0], h_full[:, :_N - 2], ec_pad[:, 0, :_NC]


# speedup vs baseline: 1.0071x; 1.0071x over previous
import jax
import jax.numpy as jnp
from jax.experimental import pallas as pl
from jax.experimental.pallas import tpu as pltpu

_B = 4
_N = 514
_P = 16
_HID = 64
_NH = 4
_HD = 16
_NC = 3
_ALPHA = 0.5
_NEG = -1e9


def _enc_kernel(nf_ref, edges_ref, wemb_ref,
                wkt1_ref, asrc1_ref, adstT1_ref, wo1_ref,
                wkt2_ref, asrc2_ref, adstT2_ref, wo2_ref,
                h_ref, hm_ref, ec_ref):
    x = nf_ref[0]
    h = jnp.dot(x, wemb_ref[...], preferred_element_type=jnp.float32)
    maskf = (edges_ref[0] > 0).astype(jnp.bfloat16)

    ones_col = jnp.ones((_N, 1), jnp.float32)

    def gcrn(h, wkt_ref, asrc_ref, adstT_ref, wo_ref):
        for layer in range(2):
            acc = jnp.zeros((_N, _HID), jnp.float32)
            for c in range(_NC):
                hp = jnp.dot(h, wkt_ref[c],
                             preferred_element_type=jnp.float32)
                hpT = hp.T
                s_src = jnp.dot(hp, asrc_ref[c],
                                preferred_element_type=jnp.float32)
                s_dst = jnp.dot(adstT_ref[c], hpT,
                                preferred_element_type=jnp.float32)
                gc = jnp.exp(0.8 * s_src).astype(jnp.bfloat16)
                e1r = jnp.exp(s_dst).astype(jnp.bfloat16)
                e2r = jnp.exp(0.2 * s_dst).astype(jnp.bfloat16)
                hp_aug = jnp.concatenate([hp, ones_col],
                                         axis=1).astype(jnp.bfloat16)
                outs = []
                for hh in range(_NH):
                    m1 = gc[:, hh:hh + 1] * e1r[hh:hh + 1, :]
                    e = jnp.maximum(m1, e2r[hh:hh + 1, :]) * maskf[c]
                    oo = jnp.dot(e, hp_aug,
                                 preferred_element_type=jnp.float32)
                    den = oo[:, _HID:_HID + 1]
                    recip = jnp.where(den > 0, 1.0 / den, 0.0)
                    outs.append(oo[:, hh * _HD:(hh + 1) * _HD] * recip)
                acc = acc + jnp.concatenate(outs, axis=1)
            o = jnp.dot(acc * (1.0 / _NC), wo_ref[...],
                        preferred_element_type=jnp.float32)
            h = _ALPHA * h + (1.0 - _ALPHA) * jnp.maximum(o, 0.0)
        return h

    h = gcrn(h, wkt1_ref, asrc1_ref, adstT1_ref, wo1_ref)
    h = gcrn(h, wkt2_ref, asrc2_ref, adstT2_ref, wo2_ref)

    ec_sums = [jnp.sum(jnp.max(maskf[c], axis=1, keepdims=True)
                       .astype(jnp.float32)) * float(_NH)
               for c in range(_NC)]

    h_ref[0] = h
    hm_ref[0] = jnp.mean(h, axis=0, keepdims=True)
    lane = jax.lax.broadcasted_iota(jnp.int32, (1, 128), 1)
    scale = 1.0 / (_NH * _N * _N)
    row = jnp.zeros((1, 128), jnp.float32)
    for c in range(_NC):
        row = row + jnp.where(lane == c, ec_sums[c] * scale, 0.0)
    ec_ref[0] = row


def _prep(Wk, a):
    eye = jnp.eye(_NH, dtype=jnp.float32)
    wkt = Wk.transpose(0, 2, 1, 3).reshape(_NC, _HID, _NH * _HD)
    asrc = (a[..., :_HD][:, :, :, None] * eye[:, None, :]).reshape(
        _NC, _NH * _HD, _NH)
    adstT = (a[..., _HD:][:, :, None, :] * eye[:, :, None]).reshape(
        _NC, _NH, _NH * _HD)
    return wkt, asrc, adstT


@jax.jit
def kernel(node_features, heterogeneous_edges, W_emb, Wk1, a1, Wo1,
           Wk2, a2, Wo2):
    wkt1, asrc1, adstT1 = _prep(Wk1, a1)
    wkt2, asrc2, adstT2 = _prep(Wk2, a2)

    full3 = lambda b: (0, 0, 0)
    full2 = lambda b: (0, 0)
    h_full, hm, ec_pad = pl.pallas_call(
        _enc_kernel,
        grid=(_B,),
        in_specs=[
            pl.BlockSpec((1, _N, _P), lambda b: (b, 0, 0)),
            pl.BlockSpec((1, _NC, _N, _N), lambda b: (b, 0, 0, 0)),
            pl.BlockSpec((_P, _HID), full2),
            pl.BlockSpec((_NC, _HID, _NH * _HD), full3),
            pl.BlockSpec((_NC, _NH * _HD, _NH), full3),
            pl.BlockSpec((_NC, _NH, _NH * _HD), full3),
            pl.BlockSpec((_HID, _HID), full2),
            pl.BlockSpec((_NC, _HID, _NH * _HD), full3),
            pl.BlockSpec((_NC, _NH * _HD, _NH), full3),
            pl.BlockSpec((_NC, _NH, _NH * _HD), full3),
            pl.BlockSpec((_HID, _HID), full2),
        ],
        out_specs=[
            pl.BlockSpec((1, _N, _HID), lambda b: (b, 0, 0)),
            pl.BlockSpec((1, 1, _HID), lambda b: (b, 0, 0)),
            pl.BlockSpec((1, 1, 128), lambda b: (b, 0, 0)),
        ],
        out_shape=[
            jax.ShapeDtypeStruct((_B, _N, _HID), jnp.float32),
            jax.ShapeDtypeStruct((_B, 1, _HID), jnp.float32),
            jax.ShapeDtypeStruct((_B, 1, 128), jnp.float32),
        ],
        compiler_params=pltpu.CompilerParams(
            dimension_semantics=("parallel",)),
    )(node_features, heterogeneous_edges, W_emb,
      wkt1, asrc1, adstT1, Wo1, wkt2, asrc2, adstT2, Wo2)

    return hm[:, 0], h_full[:, :_N - 2], ec_pad[:, 0, :_NC]
